# baseline (speedup 1.0000x reference)
import functools

import jax
import jax.numpy as jnp
from jax import lax
from jax.experimental import pallas as pl
from jax.experimental.pallas import tpu as pltpu
from jax.experimental.pallas import tpu_sc as plsc

_NW = 32
_CB = 512


@functools.lru_cache(maxsize=None)
def _make_gather(B, H, V, D):
    n_d8 = D // 8
    slab = (B // 128) * 1024
    wslab = 8 * _CB
    mesh = plsc.VectorSubcoreMesh(core_axis_name="c", subcore_axis_name="s")

    scratch = (
        [pltpu.VMEM((_CB,), jnp.int32) for _ in range(2)]
        + [pltpu.VMEM((_CB, D), jnp.float32) for _ in range(2)]
        + [pltpu.VMEM((n_d8 * wslab,), jnp.float32) for _ in range(2)]
        + [pltpu.SemaphoreType.DMA for _ in range(6)]
    )

    @functools.partial(
        pl.kernel,
        out_type=jax.ShapeDtypeStruct((H, n_d8, slab), jnp.float32),
        mesh=mesh,
        scratch_types=scratch,
        compiler_params=pltpu.CompilerParams(
            use_tc_tiling_on_sc=False, needs_layout_passes=False),
    )
    def gather_kernel(ids_hbm, table_hbm, out_hbm, *bufs):
        idx_v = bufs[0:2]
        rows_v = bufs[2:4]
        tbuf_v = bufs[4:6]
        sem_i = bufs[6:8]
        sem_g = bufs[8:10]
        sem_o = bufs[10:12]

        wid = lax.axis_index("s") * 2 + lax.axis_index("c")
        b0 = wid * _CB
        lane = lax.iota(jnp.int32, 16)

        dv0 = lane
        dv1 = lane + 16
        svec0 = lax.shift_right_logical(dv0, 3) * wslab + (dv0 & 7) * 128
        svec1 = lax.shift_right_logical(dv1, 3) * wslab + (dv1 & 7) * 128

        def transpose(rows, tbuf):
            def tr_body(j, carry):
                t_loc = lax.shift_right_logical(j, 7)
                c = j & 127
                base = t_loc * 1024 + c
                tbv0 = svec0 + base
                tbv1 = svec1 + base
                plsc.store_scatter(tbuf, [tbv0], rows[j, pl.ds(0, 16)])
                plsc.store_scatter(tbuf, [tbv1], rows[j, pl.ds(16, 16)])
                return carry

            lax.fori_loop(0, _CB, tr_body, 0)

        def fire_idx(g, b):
            pltpu.async_copy(ids_hbm.at[g, pl.ds(b0, _CB)], idx_v[b], sem_i[b])

        def wait_idx(g, b):
            pltpu.make_async_copy(
                ids_hbm.at[g, pl.ds(b0, _CB)], idx_v[b], sem_i[b]).wait()

        def fire_writeback(g, b):
            for d8 in range(n_d8):
                pltpu.async_copy(
                    tbuf_v[b].at[pl.ds(d8 * wslab, wslab)],
                    out_hbm.at[g, d8, pl.ds(wid * wslab, wslab)], sem_o[b])

        def wait_writeback(g, b):
            for d8 in range(n_d8):
                pltpu.make_async_copy(
                    tbuf_v[b].at[pl.ds(d8 * wslab, wslab)],
                    out_hbm.at[g, d8, pl.ds(wid * wslab, wslab)],
                    sem_o[b]).wait()

        fire_idx(0, 0)
        fire_idx(1, 1)

        def step(t, carry):
            for bi in range(2):
                g = 2 * t + bi
                b = bi
                b2 = 1 - bi

                @pl.when(g < H)
                def _():
                    wait_idx(g, b)
                    pltpu.async_copy(
                        table_hbm.at[idx_v[b]], rows_v[b], sem_g[b])

                @pl.when(jnp.logical_and(g >= 1, g <= H))
                def _():
                    pltpu.make_async_copy(
                        table_hbm.at[idx_v[b2]], rows_v[b2], sem_g[b2]).wait()

                    @pl.when(g + 1 < H)
                    def _():
                        fire_idx(g + 1, b2)

                    @pl.when(g >= 3)
                    def _():
                        wait_writeback(g - 3, b2)

                    transpose(rows_v[b2], tbuf_v[b2])
                    fire_writeback(g - 1, b2)
            return carry

        lax.fori_loop(0, H // 2 + 1, step, 0)

        wait_writeback(H - 2, H % 2)
        wait_writeback(H - 1, 1 - H % 2)

    return gather_kernel


def kernel(ids, table):
    B, H = ids.shape
    V, D = table.shape
    ids_t = jnp.transpose(ids).astype(jnp.int32)
    out5 = _make_gather(B, H, V, D)(ids_t, table)
    out5 = out5.reshape(H, D // 8, B // 128, 8, 128)
    x = jnp.transpose(out5, (2, 4, 0, 1, 3))
    return x.reshape(B, H, D)


# speedup vs baseline: 1.9486x; 1.9486x over previous
import functools

import jax
import jax.numpy as jnp
from jax import lax
from jax.experimental import pallas as pl
from jax.experimental.pallas import tpu as pltpu
from jax.experimental.pallas import tpu_sc as plsc

_NW = 32
_CB = 512


@functools.lru_cache(maxsize=None)
def _make_gather(B, H, V, D):
    n_d8 = D // 8
    slab = (B // 128) * 1024
    wslab = 8 * _CB
    mesh = plsc.VectorSubcoreMesh(core_axis_name="c", subcore_axis_name="s")

    scratch = (
        [pltpu.VMEM((_CB,), jnp.int32) for _ in range(2)]
        + [pltpu.VMEM((_CB, D), jnp.float32) for _ in range(2)]
        + [pltpu.VMEM((n_d8 * wslab,), jnp.float32) for _ in range(2)]
        + [pltpu.SemaphoreType.DMA for _ in range(6)]
    )

    @functools.partial(
        pl.kernel,
        out_type=jax.ShapeDtypeStruct((H, n_d8, slab), jnp.float32),
        mesh=mesh,
        scratch_types=scratch,
        compiler_params=pltpu.CompilerParams(
            use_tc_tiling_on_sc=False, needs_layout_passes=False),
    )
    def gather_kernel(ids_hbm, table_hbm, out_hbm, *bufs):
        idx_v = bufs[0:2]
        rows_v = bufs[2:4]
        tbuf_v = bufs[4:6]
        sem_i = bufs[6:8]
        sem_g = bufs[8:10]
        sem_o = bufs[10:12]

        wid = lax.axis_index("s") * 2 + lax.axis_index("c")
        b0 = wid * _CB
        lane = lax.iota(jnp.int32, 16)

        dvecs = [(d0 + lane) & (D - 1) for d0 in range(D)]
        svecs = [
            lax.shift_right_logical(dv, 3) * wslab + (dv & 7) * 128 + lane
            for dv in dvecs
        ]

        def transpose(rows, tbuf):
            def jg_body(jb, carry):
                jvec = jb * 16 + lane
                obase = lax.shift_right_logical(jb, 3) * 1024 + (jb & 7) * 16
                for d0 in range(D):
                    v = plsc.load_gather(rows, [jvec, dvecs[d0]])
                    plsc.store_scatter(tbuf, [svecs[d0] + obase], v)
                return carry

            lax.fori_loop(0, _CB // 16, jg_body, 0)

        def fire_idx(g, b):
            pltpu.async_copy(ids_hbm.at[g, pl.ds(b0, _CB)], idx_v[b], sem_i[b])

        def wait_idx(g, b):
            pltpu.make_async_copy(
                ids_hbm.at[g, pl.ds(b0, _CB)], idx_v[b], sem_i[b]).wait()

        def fire_gather(b):
            pltpu.async_copy(table_hbm.at[idx_v[b]], rows_v[b], sem_g[b])

        def wait_gather(b):
            pltpu.make_async_copy(
                table_hbm.at[idx_v[b]], rows_v[b], sem_g[b]).wait()

        def fire_writeback(g, b):
            for d8 in range(n_d8):
                pltpu.async_copy(
                    tbuf_v[b].at[pl.ds(d8 * wslab, wslab)],
                    out_hbm.at[g, d8, pl.ds(wid * wslab, wslab)], sem_o[b])

        def wait_writeback(g, b):
            for d8 in range(n_d8):
                pltpu.make_async_copy(
                    tbuf_v[b].at[pl.ds(d8 * wslab, wslab)],
                    out_hbm.at[g, d8, pl.ds(wid * wslab, wslab)],
                    sem_o[b]).wait()

        fire_idx(0, 0)
        fire_idx(1, 1)

        def step(t, carry):
            for bi in range(2):
                g = 2 * t + bi
                b = bi
                b2 = 1 - bi

                @pl.when(g < H)
                def _():
                    wait_idx(g, b)
                    fire_gather(b)

                @pl.when(jnp.logical_and(g >= 1, g <= H))
                def _():
                    wait_gather(b2)

                    @pl.when(g + 1 < H)
                    def _():
                        fire_idx(g + 1, b2)

                    @pl.when(g >= 3)
                    def _():
                        wait_writeback(g - 3, b2)

                    transpose(rows_v[b2], tbuf_v[b2])
                    fire_writeback(g - 1, b2)
            return carry

        lax.fori_loop(0, H // 2 + 1, step, 0)

        wait_writeback(H - 2, H % 2)
        wait_writeback(H - 1, 1 - H % 2)

    return gather_kernel


def kernel(ids, table):
    B, H = ids.shape
    V, D = table.shape
    ids_t = jnp.transpose(ids).astype(jnp.int32)
    out5 = _make_gather(B, H, V, D)(ids_t, table)
    out5 = out5.reshape(H, D // 8, B // 128, 8, 128)
    x = jnp.transpose(out5, (2, 4, 0, 1, 3))
    return x.reshape(B, H, D)
